# baseline (speedup 1.0000x reference)
import functools

import jax
import jax.numpy as jnp
from jax import lax
from jax.experimental import pallas as pl
from jax.experimental.pallas import tpu as pltpu
from jax.experimental.pallas import tpu_sc as plsc

_N = 442368
_CIN = 75
_CH = 24
_GRID = 96
_NVOX = _GRID * _GRID * 128


_BNL = 36864


def _lin_body(x_ref, y_ref, z_ref, lin_ref):
    lin_ref[...] = (x_ref[...] * _GRID + y_ref[...]) * 128 + z_ref[...]


def _lin_call(xs, ys, zs):
    spec = pl.BlockSpec((_BNL,), lambda i: (i,))
    return pl.pallas_call(
        _lin_body,
        grid=(_N // _BNL,),
        in_specs=[spec, spec, spec],
        out_specs=[spec],
        out_shape=[jax.ShapeDtypeStruct((_N,), jnp.int32)],
    )(xs, ys, zs)[0]


def _dense_body(ft_ref, w1_ref, b1_ref, wto_ref, bto_ref,
                pre_ref, t_ref, o_ref):
    ft = ft_ref[...]
    h = jax.lax.dot_general(w1_ref[...], ft, (((0,), (0,)), ((), ())))
    h = jnp.maximum(h + b1_ref[...], 0.0)
    to = jax.lax.dot_general(wto_ref[...], h, (((0,), (0,)), ((), ())))
    to = to + bto_ref[...]
    keep = to[1:2, :] > 0.0
    pre = jnp.concatenate([h, to], axis=0)
    pre_ref[...] = jnp.where(keep, pre, 0.0)
    t_ref[...] = to[0, :]
    o_ref[...] = to[1, :]


def _dense_call(ft, W1, b1c, wto, bto):
    grid = _N // _BNL
    full = lambda i: (0, 0)
    return pl.pallas_call(
        _dense_body,
        grid=(grid,),
        in_specs=[
            pl.BlockSpec((_CIN, _BNL), lambda i: (0, i)),
            pl.BlockSpec((_CIN, _CH), full),
            pl.BlockSpec((_CH, 1), full),
            pl.BlockSpec((_CH, 2), full),
            pl.BlockSpec((2, 1), full),
        ],
        out_specs=[
            pl.BlockSpec((_CH + 2, _BNL), lambda i: (0, i)),
            pl.BlockSpec((_BNL,), lambda i: (i,)),
            pl.BlockSpec((_BNL,), lambda i: (i,)),
        ],
        out_shape=[
            jax.ShapeDtypeStruct((_CH + 2, _N), jnp.float32),
            jax.ShapeDtypeStruct((_N,), jnp.float32),
            jax.ShapeDtypeStruct((_N,), jnp.float32),
        ],
    )(ft, W1, b1c, wto, bto)


_NC = 2
_NS = 16
_NW = _NC * _NS
_CNK = _N // _NW
_L = 16


_VSL = _NVOX // _NS


def _gather_body(idx_hbm, vols_hbm, out_hbm,
                 idx_v, t_v, vol_sp, sem_ld, sem_g):
    cid = lax.axis_index("c")
    sid = lax.axis_index("s")
    vb = sid * _VSL
    ld = pltpu.async_copy(vols_hbm.at[cid, pl.ds(vb, _VSL)],
                          vol_sp.at[pl.ds(vb, _VSL)], sem_ld)
    b0 = sid * 2 * _CNK

    def _chunk(base, first):
        pltpu.sync_copy(idx_hbm.at[pl.ds(base, _CNK)], idx_v)
        if first:
            ld.wait()
            plsc.subcore_barrier()
        pltpu.async_copy(vol_sp.at[idx_v], t_v, sem_g).wait()
        pltpu.sync_copy(t_v, out_hbm.at[cid, pl.ds(base, _CNK)])

    _chunk(b0, True)
    _chunk(b0 + _CNK, False)


@functools.cache
def _make_gather_call():
    return pl.kernel(
        _gather_body,
        out_type=jax.ShapeDtypeStruct((2, _N), jnp.float32),
        mesh=plsc.VectorSubcoreMesh(core_axis_name="c", subcore_axis_name="s"),
        scratch_types=[
            pltpu.VMEM((_CNK,), jnp.int32),
            pltpu.VMEM((_CNK,), jnp.float32),
            pltpu.VMEM_SHARED((_NVOX,), jnp.float32),
            pltpu.SemaphoreType.DMA,
            pltpu.SemaphoreType.DMA,
        ],
    )


def kernel(feat, coords, batch_idx, tsdf_vol, occ_vol, W1, b1, Wt, bt, Wo, bo):
    del batch_idx
    lin = _lin_call(coords[:, 0], coords[:, 1], coords[:, 2])
    zpad = ((0, 0), (0, 0), (0, 128 - _GRID))
    vols = jnp.pad(jnp.stack([tsdf_vol.reshape(_GRID * _GRID, _GRID),
                              occ_vol.reshape(_GRID * _GRID, _GRID)]),
                   zpad).reshape(2, _NVOX)
    out2 = _make_gather_call()(lin, vols)
    tt, ot = out2[0], out2[1]
    wto = jnp.concatenate([Wt, Wo], axis=1)
    bto = jnp.concatenate([bt, bo]).reshape(2, 1)
    pre_t, t1, o1 = _dense_call(feat.T, W1, b1.reshape(_CH, 1), wto, bto)
    return (t1.reshape(_N, 1), o1.reshape(_N, 1), tt, ot, pre_t.T)


# speedup vs baseline: 1.0592x; 1.0592x over previous
import functools

import jax
import jax.numpy as jnp
from jax import lax
from jax.experimental import pallas as pl
from jax.experimental.pallas import tpu as pltpu
from jax.experimental.pallas import tpu_sc as plsc

_N = 442368
_CIN = 75
_CH = 24
_GRID = 96
_NVOX = _GRID * _GRID * 128


_BNL = 18432


def _dense_body(ft_ref, w1_ref, b1_ref, wto_ref, bto_ref,
                pre_ref, t_ref, o_ref):
    ft = ft_ref[...]
    h = jax.lax.dot_general(w1_ref[...], ft, (((0,), (0,)), ((), ())))
    h = jnp.maximum(h + b1_ref[...], 0.0)
    to = jax.lax.dot_general(wto_ref[...], h, (((0,), (0,)), ((), ())))
    to = to + bto_ref[...]
    keep = to[1:2, :] > 0.0
    pre = jnp.concatenate([h, to], axis=0)
    pre_ref[...] = jnp.where(keep, pre, 0.0)
    t_ref[...] = to[0, :]
    o_ref[...] = to[1, :]


def _dense_call(ft, W1, b1c, wto, bto):
    grid = _N // _BNL
    full = lambda i: (0, 0)
    return pl.pallas_call(
        _dense_body,
        grid=(grid,),
        in_specs=[
            pl.BlockSpec((_CIN, _BNL), lambda i: (0, i)),
            pl.BlockSpec((_CIN, _CH), full),
            pl.BlockSpec((_CH, 1), full),
            pl.BlockSpec((_CH, 2), full),
            pl.BlockSpec((2, 1), full),
        ],
        out_specs=[
            pl.BlockSpec((_CH + 2, _BNL), lambda i: (0, i)),
            pl.BlockSpec((_BNL,), lambda i: (i,)),
            pl.BlockSpec((_BNL,), lambda i: (i,)),
        ],
        out_shape=[
            jax.ShapeDtypeStruct((_CH + 2, _N), jnp.float32),
            jax.ShapeDtypeStruct((_N,), jnp.float32),
            jax.ShapeDtypeStruct((_N,), jnp.float32),
        ],
    )(ft, W1, b1c, wto, bto)


_NC = 2
_NS = 16
_NW = _NC * _NS
_CNK = _N // _NW
_L = 16


_VSL = _NVOX // _NS


def _gather_body(idx_hbm, vols_hbm, out_hbm,
                 idx_v, t_v, vol_sp, sem_ld, sem_g):
    cid = lax.axis_index("c")
    sid = lax.axis_index("s")
    vb = sid * _VSL
    ld = pltpu.async_copy(vols_hbm.at[cid, pl.ds(vb, _VSL)],
                          vol_sp.at[pl.ds(vb, _VSL)], sem_ld)
    b0 = sid * 2 * _CNK

    def _chunk(base, first):
        pltpu.sync_copy(idx_hbm.at[pl.ds(base, _CNK)], idx_v)
        if first:
            ld.wait()
            plsc.subcore_barrier()
        pltpu.async_copy(vol_sp.at[idx_v], t_v, sem_g).wait()
        pltpu.sync_copy(t_v, out_hbm.at[cid, pl.ds(base, _CNK)])

    _chunk(b0, True)
    _chunk(b0 + _CNK, False)


@functools.cache
def _make_gather_call():
    return pl.kernel(
        _gather_body,
        out_type=jax.ShapeDtypeStruct((2, _N), jnp.float32),
        mesh=plsc.VectorSubcoreMesh(core_axis_name="c", subcore_axis_name="s"),
        scratch_types=[
            pltpu.VMEM((_CNK,), jnp.int32),
            pltpu.VMEM((_CNK,), jnp.float32),
            pltpu.VMEM_SHARED((_NVOX,), jnp.float32),
            pltpu.SemaphoreType.DMA,
            pltpu.SemaphoreType.DMA,
        ],
    )


def kernel(feat, coords, batch_idx, tsdf_vol, occ_vol, W1, b1, Wt, bt, Wo, bo):
    del batch_idx
    lin = (coords[:, 0] * _GRID + coords[:, 1]) * 128 + coords[:, 2]
    zpad = ((0, 0), (0, 0), (0, 128 - _GRID))
    vols = jnp.pad(jnp.stack([tsdf_vol.reshape(_GRID * _GRID, _GRID),
                              occ_vol.reshape(_GRID * _GRID, _GRID)]),
                   zpad).reshape(2, _NVOX)
    out2 = _make_gather_call()(lin, vols)
    tt, ot = out2[0], out2[1]
    wto = jnp.concatenate([Wt, Wo], axis=1)
    bto = jnp.concatenate([bt, bo]).reshape(2, 1)
    pre_t, t1, o1 = _dense_call(feat.T, W1, b1.reshape(_CH, 1), wto, bto)
    return (t1.reshape(_N, 1), o1.reshape(_N, 1), tt, ot, pre_t.T)


# speedup vs baseline: 1.0718x; 1.0119x over previous
import functools

import jax
import jax.numpy as jnp
from jax import lax
from jax.experimental import pallas as pl
from jax.experimental.pallas import tpu as pltpu
from jax.experimental.pallas import tpu_sc as plsc

_N = 442368
_CIN = 75
_CH = 24
_GRID = 96
_NVOX = _GRID * _GRID * 128


_BNL = 55296


def _dense_body(ft_ref, w1_ref, b1_ref, wto_ref, bto_ref,
                pre_ref, t_ref, o_ref):
    ft = ft_ref[...]
    h = jax.lax.dot_general(w1_ref[...], ft, (((0,), (0,)), ((), ())))
    h = jnp.maximum(h + b1_ref[...], 0.0)
    to = jax.lax.dot_general(wto_ref[...], h, (((0,), (0,)), ((), ())))
    to = to + bto_ref[...]
    keep = to[1:2, :] > 0.0
    pre = jnp.concatenate([h, to], axis=0)
    pre_ref[...] = jnp.where(keep, pre, 0.0)
    t_ref[...] = to[0, :]
    o_ref[...] = to[1, :]


def _dense_call(ft, W1, b1c, wto, bto):
    grid = _N // _BNL
    full = lambda i: (0, 0)
    return pl.pallas_call(
        _dense_body,
        grid=(grid,),
        in_specs=[
            pl.BlockSpec((_CIN, _BNL), lambda i: (0, i)),
            pl.BlockSpec((_CIN, _CH), full),
            pl.BlockSpec((_CH, 1), full),
            pl.BlockSpec((_CH, 2), full),
            pl.BlockSpec((2, 1), full),
        ],
        out_specs=[
            pl.BlockSpec((_CH + 2, _BNL), lambda i: (0, i)),
            pl.BlockSpec((_BNL,), lambda i: (i,)),
            pl.BlockSpec((_BNL,), lambda i: (i,)),
        ],
        out_shape=[
            jax.ShapeDtypeStruct((_CH + 2, _N), jnp.float32),
            jax.ShapeDtypeStruct((_N,), jnp.float32),
            jax.ShapeDtypeStruct((_N,), jnp.float32),
        ],
    )(ft, W1, b1c, wto, bto)


_NC = 2
_NS = 16
_NW = _NC * _NS
_CNK = _N // _NW
_L = 16


_VSL = _NVOX // _NS


def _gather_body(idx_hbm, vols_hbm, out_hbm,
                 idx_v, t_v, vol_sp, sem_ld, sem_g):
    cid = lax.axis_index("c")
    sid = lax.axis_index("s")
    vb = sid * _VSL
    ld = pltpu.async_copy(vols_hbm.at[cid, pl.ds(vb, _VSL)],
                          vol_sp.at[pl.ds(vb, _VSL)], sem_ld)
    b0 = sid * 2 * _CNK

    def _chunk(base, first):
        pltpu.sync_copy(idx_hbm.at[pl.ds(base, _CNK)], idx_v)
        if first:
            ld.wait()
            plsc.subcore_barrier()
        pltpu.async_copy(vol_sp.at[idx_v], t_v, sem_g).wait()
        pltpu.sync_copy(t_v, out_hbm.at[cid, pl.ds(base, _CNK)])

    _chunk(b0, True)
    _chunk(b0 + _CNK, False)


@functools.cache
def _make_gather_call():
    return pl.kernel(
        _gather_body,
        out_type=jax.ShapeDtypeStruct((2, _N), jnp.float32),
        mesh=plsc.VectorSubcoreMesh(core_axis_name="c", subcore_axis_name="s"),
        scratch_types=[
            pltpu.VMEM((_CNK,), jnp.int32),
            pltpu.VMEM((_CNK,), jnp.float32),
            pltpu.VMEM_SHARED((_NVOX,), jnp.float32),
            pltpu.SemaphoreType.DMA,
            pltpu.SemaphoreType.DMA,
        ],
    )


def kernel(feat, coords, batch_idx, tsdf_vol, occ_vol, W1, b1, Wt, bt, Wo, bo):
    del batch_idx
    lin = (coords[:, 0] * _GRID + coords[:, 1]) * 128 + coords[:, 2]
    zpad = ((0, 0), (0, 0), (0, 128 - _GRID))
    vols = jnp.pad(jnp.stack([tsdf_vol.reshape(_GRID * _GRID, _GRID),
                              occ_vol.reshape(_GRID * _GRID, _GRID)]),
                   zpad).reshape(2, _NVOX)
    out2 = _make_gather_call()(lin, vols)
    tt, ot = out2[0], out2[1]
    wto = jnp.concatenate([Wt, Wo], axis=1)
    bto = jnp.concatenate([bt, bo]).reshape(2, 1)
    pre_t, t1, o1 = _dense_call(feat.T, W1, b1.reshape(_CH, 1), wto, bto)
    return (t1.reshape(_N, 1), o1.reshape(_N, 1), tt, ot, pre_t.T)


# speedup vs baseline: 1.1052x; 1.0311x over previous
import functools

import jax
import jax.numpy as jnp
from jax import lax
from jax.experimental import pallas as pl
from jax.experimental.pallas import tpu as pltpu
from jax.experimental.pallas import tpu_sc as plsc

_N = 442368
_CIN = 75
_CH = 24
_GRID = 96
_NVOX = _GRID * _GRID * 128


_BNL = 55296


_BLIN = 36864


def _lin_body(ct_ref, lin_ref):
    c = ct_ref[...]
    lin_ref[...] = (c[0, :] * _GRID + c[1, :]) * 128 + c[2, :]


def _lin_call(ct):
    return pl.pallas_call(
        _lin_body,
        grid=(_N // _BLIN,),
        in_specs=[pl.BlockSpec((3, _BLIN), lambda i: (0, i))],
        out_specs=[pl.BlockSpec((_BLIN,), lambda i: (i,))],
        out_shape=[jax.ShapeDtypeStruct((_N,), jnp.int32)],
    )(ct)[0]


def _dense_body(ft_ref, w1_ref, b1_ref, wto_ref, bto_ref,
                pre_ref, t_ref, o_ref):
    ft = ft_ref[...]
    h = jax.lax.dot_general(w1_ref[...], ft, (((0,), (0,)), ((), ())))
    h = jnp.maximum(h + b1_ref[...], 0.0)
    to = jax.lax.dot_general(wto_ref[...], h, (((0,), (0,)), ((), ())))
    to = to + bto_ref[...]
    keep = to[1:2, :] > 0.0
    pre = jnp.concatenate([h, to], axis=0)
    pre_ref[...] = jnp.where(keep, pre, 0.0)
    t_ref[...] = to[0, :]
    o_ref[...] = to[1, :]


def _dense_call(ft, W1, b1c, wto, bto):
    grid = _N // _BNL
    full = lambda i: (0, 0)
    return pl.pallas_call(
        _dense_body,
        grid=(grid,),
        in_specs=[
            pl.BlockSpec((_CIN, _BNL), lambda i: (0, i)),
            pl.BlockSpec((_CIN, _CH), full),
            pl.BlockSpec((_CH, 1), full),
            pl.BlockSpec((_CH, 2), full),
            pl.BlockSpec((2, 1), full),
        ],
        out_specs=[
            pl.BlockSpec((_CH + 2, _BNL), lambda i: (0, i)),
            pl.BlockSpec((_BNL,), lambda i: (i,)),
            pl.BlockSpec((_BNL,), lambda i: (i,)),
        ],
        out_shape=[
            jax.ShapeDtypeStruct((_CH + 2, _N), jnp.float32),
            jax.ShapeDtypeStruct((_N,), jnp.float32),
            jax.ShapeDtypeStruct((_N,), jnp.float32),
        ],
    )(ft, W1, b1c, wto, bto)


_NC = 2
_NS = 16
_NW = _NC * _NS
_CNK = _N // _NW
_L = 16


_VSL = _NVOX // _NS


def _gather_body(idx_hbm, vols_hbm, out_hbm,
                 idx_v, t_v, vol_sp, sem_ld, sem_g):
    cid = lax.axis_index("c")
    sid = lax.axis_index("s")
    vb = sid * _VSL
    ld = pltpu.async_copy(vols_hbm.at[cid, pl.ds(vb, _VSL)],
                          vol_sp.at[pl.ds(vb, _VSL)], sem_ld)
    b0 = sid * 2 * _CNK

    def _chunk(base, first):
        pltpu.sync_copy(idx_hbm.at[pl.ds(base, _CNK)], idx_v)
        if first:
            ld.wait()
            plsc.subcore_barrier()
        pltpu.async_copy(vol_sp.at[idx_v], t_v, sem_g).wait()
        pltpu.sync_copy(t_v, out_hbm.at[cid, pl.ds(base, _CNK)])

    _chunk(b0, True)
    _chunk(b0 + _CNK, False)


@functools.cache
def _make_gather_call():
    return pl.kernel(
        _gather_body,
        out_type=jax.ShapeDtypeStruct((2, _N), jnp.float32),
        mesh=plsc.VectorSubcoreMesh(core_axis_name="c", subcore_axis_name="s"),
        scratch_types=[
            pltpu.VMEM((_CNK,), jnp.int32),
            pltpu.VMEM((_CNK,), jnp.float32),
            pltpu.VMEM_SHARED((_NVOX,), jnp.float32),
            pltpu.SemaphoreType.DMA,
            pltpu.SemaphoreType.DMA,
        ],
    )


def kernel(feat, coords, batch_idx, tsdf_vol, occ_vol, W1, b1, Wt, bt, Wo, bo):
    del batch_idx
    lin = _lin_call(coords.T)
    zpad = ((0, 0), (0, 0), (0, 128 - _GRID))
    vols = jnp.pad(jnp.stack([tsdf_vol.reshape(_GRID * _GRID, _GRID),
                              occ_vol.reshape(_GRID * _GRID, _GRID)]),
                   zpad).reshape(2, _NVOX)
    out2 = _make_gather_call()(lin, vols)
    tt, ot = out2[0], out2[1]
    wto = jnp.concatenate([Wt, Wo], axis=1)
    bto = jnp.concatenate([bt, bo]).reshape(2, 1)
    pre_t, t1, o1 = _dense_call(feat.T, W1, b1.reshape(_CH, 1), wto, bto)
    return (t1.reshape(_N, 1), o1.reshape(_N, 1), tt, ot, pre_t.T)


# speedup vs baseline: 1.1063x; 1.0011x over previous
import functools

import jax
import jax.numpy as jnp
from jax import lax
from jax.experimental import pallas as pl
from jax.experimental.pallas import tpu as pltpu
from jax.experimental.pallas import tpu_sc as plsc

_N = 442368
_CIN = 75
_CH = 24
_GRID = 96
_NVOX = _GRID * _GRID * 128


_BNL = 36864


_BLIN = 36864


def _lin_body(ct_ref, lin_ref):
    c = ct_ref[...]
    lin_ref[...] = (c[0, :] * _GRID + c[1, :]) * 128 + c[2, :]


def _lin_call(ct):
    return pl.pallas_call(
        _lin_body,
        grid=(_N // _BLIN,),
        in_specs=[pl.BlockSpec((3, _BLIN), lambda i: (0, i))],
        out_specs=[pl.BlockSpec((_BLIN,), lambda i: (i,))],
        out_shape=[jax.ShapeDtypeStruct((_N,), jnp.int32)],
    )(ct)[0]


def _dense_body(ft_ref, w1_ref, b1_ref, wto_ref, bto_ref,
                pre_ref, t_ref, o_ref):
    ft = ft_ref[...]
    h = jax.lax.dot_general(w1_ref[...], ft, (((0,), (0,)), ((), ())))
    h = jnp.maximum(h + b1_ref[...], 0.0)
    to = jax.lax.dot_general(wto_ref[...], h, (((0,), (0,)), ((), ())))
    to = to + bto_ref[...]
    keep = to[1:2, :] > 0.0
    pre = jnp.concatenate([h, to], axis=0)
    pre_ref[...] = jnp.where(keep, pre, 0.0)
    t_ref[...] = to[0, :]
    o_ref[...] = to[1, :]


def _dense_call(ft, W1, b1c, wto, bto):
    grid = _N // _BNL
    full = lambda i: (0, 0)
    return pl.pallas_call(
        _dense_body,
        grid=(grid,),
        in_specs=[
            pl.BlockSpec((_CIN, _BNL), lambda i: (0, i)),
            pl.BlockSpec((_CIN, _CH), full),
            pl.BlockSpec((_CH, 1), full),
            pl.BlockSpec((_CH, 2), full),
            pl.BlockSpec((2, 1), full),
        ],
        out_specs=[
            pl.BlockSpec((_CH + 2, _BNL), lambda i: (0, i)),
            pl.BlockSpec((_BNL,), lambda i: (i,)),
            pl.BlockSpec((_BNL,), lambda i: (i,)),
        ],
        out_shape=[
            jax.ShapeDtypeStruct((_CH + 2, _N), jnp.float32),
            jax.ShapeDtypeStruct((_N,), jnp.float32),
            jax.ShapeDtypeStruct((_N,), jnp.float32),
        ],
        compiler_params=pltpu.CompilerParams(
            dimension_semantics=("parallel",)),
    )(ft, W1, b1c, wto, bto)


_NC = 2
_NS = 16
_NW = _NC * _NS
_CNK = _N // _NW
_L = 16


_VSL = _NVOX // _NS


def _gather_body(idx_hbm, vols_hbm, out_hbm,
                 idx_v, t_v, vol_sp, sem_ld, sem_g):
    cid = lax.axis_index("c")
    sid = lax.axis_index("s")
    vb = sid * _VSL
    ld = pltpu.async_copy(vols_hbm.at[cid, pl.ds(vb, _VSL)],
                          vol_sp.at[pl.ds(vb, _VSL)], sem_ld)
    b0 = sid * 2 * _CNK

    def _chunk(base, first):
        pltpu.sync_copy(idx_hbm.at[pl.ds(base, _CNK)], idx_v)
        if first:
            ld.wait()
            plsc.subcore_barrier()
        pltpu.async_copy(vol_sp.at[idx_v], t_v, sem_g).wait()
        pltpu.sync_copy(t_v, out_hbm.at[cid, pl.ds(base, _CNK)])

    _chunk(b0, True)
    _chunk(b0 + _CNK, False)


@functools.cache
def _make_gather_call():
    return pl.kernel(
        _gather_body,
        out_type=jax.ShapeDtypeStruct((2, _N), jnp.float32),
        mesh=plsc.VectorSubcoreMesh(core_axis_name="c", subcore_axis_name="s"),
        scratch_types=[
            pltpu.VMEM((_CNK,), jnp.int32),
            pltpu.VMEM((_CNK,), jnp.float32),
            pltpu.VMEM_SHARED((_NVOX,), jnp.float32),
            pltpu.SemaphoreType.DMA,
            pltpu.SemaphoreType.DMA,
        ],
    )


def kernel(feat, coords, batch_idx, tsdf_vol, occ_vol, W1, b1, Wt, bt, Wo, bo):
    del batch_idx
    lin = _lin_call(coords.T)
    zpad = ((0, 0), (0, 0), (0, 128 - _GRID))
    vols = jnp.pad(jnp.stack([tsdf_vol.reshape(_GRID * _GRID, _GRID),
                              occ_vol.reshape(_GRID * _GRID, _GRID)]),
                   zpad).reshape(2, _NVOX)
    out2 = _make_gather_call()(lin, vols)
    tt, ot = out2[0], out2[1]
    wto = jnp.concatenate([Wt, Wo], axis=1)
    bto = jnp.concatenate([bt, bo]).reshape(2, 1)
    pre_t, t1, o1 = _dense_call(feat.T, W1, b1.reshape(_CH, 1), wto, bto)
    return (t1.reshape(_N, 1), o1.reshape(_N, 1), tt, ot, pre_t.T)
